# Initial kernel scaffold; baseline (speedup 1.0000x reference)
import jax
import jax.numpy as jnp
from jax.experimental import pallas as pl


def kernel(h, edge_index, Wmsg_f, bmsg_f, Wih_f, Whh_f, bih_f, bhh_f, Wmsg_b, bmsg_b, Wih_b, Whh_b, bih_b, bhh_b):
    raise NotImplementedError("write your pallas kernel here")


# speedup vs baseline: 3.8625x; 3.8625x over previous
import functools

import jax
import jax.numpy as jnp
from jax import lax
from jax.experimental import pallas as pl
from jax.experimental.pallas import tpu as pltpu
from jax.experimental.pallas import tpu_sc as plsc

NC = 2
NS = 16
CHUNK = 128


def _make_sc_pass(np_rows: int, nch: int, trash: int):
    wl = 128
    nst = nch + 4
    stripe = np_rows // NS
    n_zero = stripe // CHUNK

    mesh = plsc.VectorSubcoreMesh(
        core_axis_name="c", subcore_axis_name="s", num_cores=NC,
        num_subcores=NS)

    def body(hf, hb, cf3d, cb3d, tf, tb,
             idx_v, rows_v, sp_idx, sp_rows, acc_s,
             semg0, semg1, semi0, semi1, semi2, semi3):
        c = lax.axis_index("c")
        s = lax.axis_index("s")
        base = s * stripe
        semg = (semg0, semg1)
        semi = (semi0, semi1, semi2, semi3)
        lanes = lax.broadcasted_iota(jnp.int32, (16,), 0)

        zv = jnp.zeros((16,), jnp.float32)

        def zrow(i, _):
            for j in range(wl // 16):
                rows_v[0, i, pl.ds(j * 16, 16)] = zv
            return 0
        lax.fori_loop(0, CHUNK, zrow, 0)
        for k in range(n_zero):
            pltpu.sync_copy(rows_v.at[0],
                            acc_s.at[pl.ds(base + k * CHUNK, CHUNK)])
        sp_idx[...] = jnp.full((16,), trash, jnp.int32)

        def zsp(i, _):
            for j in range(wl // 16):
                sp_rows[i, pl.ds(j * 16, 16)] = zv
            return 0
        lax.fori_loop(0, 16, zsp, 0)
        plsc.subcore_barrier()

        def run(htab, c3d, t_out):
            row0 = s * nst

            def fire_idx(ch, r):
                pltpu.async_copy(c3d.at[row0 + ch], idx_v.at[r], semi[r])

            def wait_idx(ch, r):
                pltpu.make_async_copy(c3d.at[row0 + ch], idx_v.at[r],
                                      semi[r]).wait()

            def fire_g(r, b):
                pltpu.async_copy(htab.at[idx_v.at[r, 0]], rows_v.at[b],
                                 semg[b])

            def wait_g(r, b):
                pltpu.make_async_copy(htab.at[idx_v.at[r, 0]], rows_v.at[b],
                                      semg[b]).wait()

            for u in range(4):
                fire_idx(u, u)
            for u in range(2):
                wait_idx(u, u)
                fire_g(u, u)

            def spill(u, b, o, hit0, v0):
                def sbody(carry):
                    hit, v = carry
                    lane = jnp.max(plsc.all_reduce_ffs(hit))
                    tval = jnp.max(jnp.where(lanes == lane, v,
                                             jnp.int32(-2147483648)))
                    pos = o + lane
                    sp_idx[...] = jnp.where(lanes == 0, tval,
                                            jnp.int32(trash))
                    for j in range(wl // 16):
                        sp_rows[0, pl.ds(j * 16, 16)] = (
                            rows_v[b, pos, pl.ds(j * 16, 16)])
                    pltpu.sync_copy(sp_rows, acc_s.at[sp_idx], add=True)
                    return (hit & (lanes != lane),
                            jnp.where(lanes == lane, jnp.int32(trash), v))

                def scond(carry):
                    return jnp.any(carry[0])

                _, vf = lax.while_loop(scond, sbody, (hit0, v0))
                idx_v[u, 1, pl.ds(o, 16)] = vf

            def dedup(u, b):
                for w in range(15):
                    o = 8 * w
                    v = idx_v[u, 1, pl.ds(o, 16)]
                    cnt, _ = plsc.scan_count(v)
                    hit = (cnt != jnp.min(cnt)) & (v != trash)

                    @pl.when(jnp.any(hit))
                    def _():
                        spill(u, b, o, hit, v)

            def step(q, _):
                cb = 4 * q
                for u in range(4):
                    ch = cb + u
                    b = u % 2
                    wait_g(u, b)
                    dedup(u, b)
                    pltpu.sync_copy(rows_v.at[b], acc_s.at[idx_v.at[u, 1]],
                                    add=True)
                    fire_idx(ch + 4, u)
                    wait_idx(ch + 2, (u + 2) % 4)
                    fire_g((u + 2) % 4, b)
                return 0
            lax.fori_loop(0, nch // 4, step, 0)
            wait_g(0, 0)
            wait_g(1, 1)
            wait_idx(nch + 2, 2)
            wait_idx(nch + 3, 3)

            plsc.subcore_barrier()
            pltpu.sync_copy(acc_s.at[pl.ds(base, stripe)],
                            t_out.at[pl.ds(base, stripe)])

        @pl.when(c == 0)
        def _():
            run(hf, cf3d, tf)

        @pl.when(c == 1)
        def _():
            run(hb, cb3d, tb)

    out = jax.ShapeDtypeStruct((np_rows, wl), jnp.float32)
    return pl.kernel(
        body,
        out_type=(out, out),
        mesh=mesh,
        compiler_params=pltpu.CompilerParams(needs_layout_passes=False),
        scratch_types=[
            pltpu.VMEM((4, 2, CHUNK), jnp.int32),
            pltpu.VMEM((2, CHUNK, wl), jnp.float32),
            pltpu.VMEM((16,), jnp.int32),
            pltpu.VMEM((16, wl), jnp.float32),
            pltpu.VMEM_SHARED((np_rows, wl), jnp.float32),
            pltpu.SemaphoreType.DMA,
            pltpu.SemaphoreType.DMA,
            pltpu.SemaphoreType.DMA,
            pltpu.SemaphoreType.DMA,
            pltpu.SemaphoreType.DMA,
            pltpu.SemaphoreType.DMA,
        ],
    )


def _stage_body(hh, t, d, w1t, w2t, bm, wir, wiz, win,
                whr, whz, whn, bir, biz, binn, bhr, bhz, bhn, out):
    hv = hh[:, :64]
    feats = t[:, :64]
    deg = d[:, 64:65]
    aggr = (jnp.dot(feats, w1t[...], preferred_element_type=jnp.float32)
            + deg * (jnp.dot(hv, w2t[...],
                             preferred_element_type=jnp.float32) + bm[...]))
    gir = jnp.dot(aggr, wir[...], preferred_element_type=jnp.float32) + bir[...]
    giz = jnp.dot(aggr, wiz[...], preferred_element_type=jnp.float32) + biz[...]
    gin = jnp.dot(aggr, win[...], preferred_element_type=jnp.float32) + binn[...]
    ghr = jnp.dot(hv, whr[...], preferred_element_type=jnp.float32) + bhr[...]
    ghz = jnp.dot(hv, whz[...], preferred_element_type=jnp.float32) + bhz[...]
    ghn = jnp.dot(hv, whn[...], preferred_element_type=jnp.float32) + bhn[...]
    r = jax.nn.sigmoid(gir + ghr)
    z = jax.nn.sigmoid(giz + ghz)
    n = jnp.tanh(gin + r * ghn)
    hn = (1.0 - z) * n + z * hv
    blk = hn.shape[0]
    out[...] = jnp.concatenate(
        [hn, jnp.ones((blk, 1), jnp.float32),
         jnp.zeros((blk, 63), jnp.float32)], axis=1)


def _make_stage(np_rows: int, blk: int = 1024):
    grid = (np_rows // blk,)
    row = lambda i: (i, 0)
    full = lambda i: (0, 0)

    def spec(shape, imap):
        return pl.BlockSpec(shape, imap)

    in_specs = [
        spec((blk, 128), row),
        spec((blk, 128), row),
        spec((blk, 128), row),
        spec((64, 128), full),
        spec((64, 128), full),
        spec((1, 128), full),
        spec((128, 64), full), spec((128, 64), full), spec((128, 64), full),
        spec((64, 64), full), spec((64, 64), full), spec((64, 64), full),
        spec((1, 64), full), spec((1, 64), full), spec((1, 64), full),
        spec((1, 64), full), spec((1, 64), full), spec((1, 64), full),
    ]
    return pl.pallas_call(
        _stage_body,
        grid=grid,
        in_specs=in_specs,
        out_specs=spec((blk, 128), row),
        out_shape=jax.ShapeDtypeStruct((np_rows, 128), jnp.float32),
    )


def _dir_weights(Wm, bm, Wih, Whh, bih, bhh, l):
    wm = Wm[l]
    w1t = wm[:, :64].T
    w2t = wm[:, 64:].T
    wih_t = Wih[l].T
    whh_t = Whh[l].T
    return (w1t, w2t, bm[l].reshape(1, 128),
            wih_t[:, :64], wih_t[:, 64:128], wih_t[:, 128:],
            whh_t[:, :64], whh_t[:, 64:128], whh_t[:, 128:],
            bih[l][:64].reshape(1, 64), bih[l][64:128].reshape(1, 64),
            bih[l][128:].reshape(1, 64),
            bhh[l][:64].reshape(1, 64), bhh[l][64:128].reshape(1, 64),
            bhh[l][128:].reshape(1, 64))


def kernel(h, edge_index, Wmsg_f, bmsg_f, Wih_f, Whh_f, bih_f, bhh_f,
           Wmsg_b, bmsg_b, Wih_b, Whh_b, bih_b, bhh_b):
    n_nodes = h.shape[0]
    e = edge_index.shape[1]
    np_rows = -(-n_nodes // 2048) * 2048

    nch = -(-e // (NS * CHUNK))
    nch = -(-nch // 4) * 4
    nst = nch + 4

    src, dst = edge_index[0], edge_index[1]
    pad = jnp.full((NS * nch * CHUNK - e,), n_nodes, jnp.int32)

    def stage_idx(x):
        x3 = jnp.concatenate([x, pad]).reshape(NS, nch, CHUNK)
        x3 = jnp.pad(x3, ((0, 0), (0, 4), (0, 0)), constant_values=n_nodes)
        return x3.reshape(NS * nst, CHUNK)

    src2d = stage_idx(src)
    dst2d = stage_idx(dst)
    comb_f = jnp.stack([src2d, dst2d], axis=1)
    comb_b = jnp.stack([dst2d, src2d], axis=1)

    def ext_table(hx):
        t = jnp.zeros((np_rows, 128), jnp.float32)
        t = t.at[:n_nodes, :64].set(hx)
        t = t.at[:n_nodes, 64].set(1.0)
        return t

    hf0 = ext_table(h[:, :64])
    hb0 = ext_table(h[:, 64:])

    sc = _make_sc_pass(np_rows, nch, n_nodes)
    stage = _make_stage(np_rows)

    t0f, t0b = sc(hf0, hb0, comb_f, comb_b)
    hf1 = stage(hf0, t0f, t0f,
                *_dir_weights(Wmsg_f, bmsg_f, Wih_f, Whh_f, bih_f, bhh_f, 0))
    hb1 = stage(hb0, t0b, t0b,
                *_dir_weights(Wmsg_b, bmsg_b, Wih_b, Whh_b, bih_b, bhh_b, 0))

    t1f, t1b = sc(hf1, hb1, comb_f, comb_b)
    hf2 = stage(hf1, t1f, t0f,
                *_dir_weights(Wmsg_f, bmsg_f, Wih_f, Whh_f, bih_f, bhh_f, 1))
    hb2 = stage(hb1, t1b, t0b,
                *_dir_weights(Wmsg_b, bmsg_b, Wih_b, Whh_b, bih_b, bhh_b, 1))

    return jnp.concatenate([hf2[:n_nodes, :64], hb2[:n_nodes, :64]], axis=1)


# speedup vs baseline: 4.1844x; 1.0833x over previous
import functools

import jax
import jax.numpy as jnp
from jax import lax
from jax.experimental import pallas as pl
from jax.experimental.pallas import tpu as pltpu
from jax.experimental.pallas import tpu_sc as plsc

NC = 2
NS = 16
CHUNK = 128


def _make_sc_pass(np_rows: int, nch: int, trash: int):
    wl = 128
    nst = nch + 4
    stripe = np_rows // NS
    n_zero = stripe // CHUNK

    mesh = plsc.VectorSubcoreMesh(
        core_axis_name="c", subcore_axis_name="s", num_cores=NC,
        num_subcores=NS)

    def body(hf, hb, cf3d, cb3d, tf, tb,
             idx_v, rows_v, acc_s,
             semg0, semg1, semi0, semi1, semi2, semi3):
        c = lax.axis_index("c")
        s = lax.axis_index("s")
        base = s * stripe
        semg = (semg0, semg1)
        semi = (semi0, semi1, semi2, semi3)

        zv = jnp.zeros((16,), jnp.float32)

        def zrow(i, _):
            for j in range(wl // 16):
                rows_v[0, i, pl.ds(j * 16, 16)] = zv
            return 0
        lax.fori_loop(0, CHUNK, zrow, 0)
        for k in range(n_zero):
            pltpu.sync_copy(rows_v.at[0],
                            acc_s.at[pl.ds(base + k * CHUNK, CHUNK)])
        plsc.subcore_barrier()

        def run(htab, c3d, t_out):
            row0 = s * nst

            def fire_idx(ch, r):
                pltpu.async_copy(c3d.at[row0 + ch], idx_v.at[r], semi[r])

            def wait_idx(ch, r):
                pltpu.make_async_copy(c3d.at[row0 + ch], idx_v.at[r],
                                      semi[r]).wait()

            def fire_g(r, b):
                pltpu.async_copy(htab.at[idx_v.at[r, 0]], rows_v.at[b],
                                 semg[b])

            def wait_g(r, b):
                pltpu.make_async_copy(htab.at[idx_v.at[r, 0]], rows_v.at[b],
                                      semg[b]).wait()

            for u in range(4):
                fire_idx(u, u)
            for u in range(2):
                wait_idx(u, u)
                fire_g(u, u)

            def step(q, _):
                cb = 4 * q
                for u in range(4):
                    ch = cb + u
                    b = u % 2
                    wait_g(u, b)
                    pltpu.sync_copy(rows_v.at[b], acc_s.at[idx_v.at[u, 1]],
                                    add=True)
                    fire_idx(ch + 4, u)
                    wait_idx(ch + 2, (u + 2) % 4)
                    fire_g((u + 2) % 4, b)
                return 0
            lax.fori_loop(0, nch // 4, step, 0)
            wait_g(0, 0)
            wait_g(1, 1)
            wait_idx(nch + 2, 2)
            wait_idx(nch + 3, 3)

            plsc.subcore_barrier()
            pltpu.sync_copy(acc_s.at[pl.ds(base, stripe)],
                            t_out.at[pl.ds(base, stripe)])

        @pl.when(c == 0)
        def _():
            run(hf, cf3d, tf)

        @pl.when(c == 1)
        def _():
            run(hb, cb3d, tb)

    out = jax.ShapeDtypeStruct((np_rows, wl), jnp.float32)
    return pl.kernel(
        body,
        out_type=(out, out),
        mesh=mesh,
        compiler_params=pltpu.CompilerParams(needs_layout_passes=False),
        scratch_types=[
            pltpu.VMEM((4, 2, CHUNK), jnp.int32),
            pltpu.VMEM((2, CHUNK, wl), jnp.float32),
            pltpu.VMEM_SHARED((np_rows, wl), jnp.float32),
            pltpu.SemaphoreType.DMA,
            pltpu.SemaphoreType.DMA,
            pltpu.SemaphoreType.DMA,
            pltpu.SemaphoreType.DMA,
            pltpu.SemaphoreType.DMA,
            pltpu.SemaphoreType.DMA,
        ],
    )


def _stage_body(hh, t, d, w1t, w2t, bm, wir, wiz, win,
                whr, whz, whn, bir, biz, binn, bhr, bhz, bhn, out):
    hv = hh[:, :64]
    feats = t[:, :64]
    deg = d[:, 64:65]
    aggr = (jnp.dot(feats, w1t[...], preferred_element_type=jnp.float32)
            + deg * (jnp.dot(hv, w2t[...],
                             preferred_element_type=jnp.float32) + bm[...]))
    gir = jnp.dot(aggr, wir[...], preferred_element_type=jnp.float32) + bir[...]
    giz = jnp.dot(aggr, wiz[...], preferred_element_type=jnp.float32) + biz[...]
    gin = jnp.dot(aggr, win[...], preferred_element_type=jnp.float32) + binn[...]
    ghr = jnp.dot(hv, whr[...], preferred_element_type=jnp.float32) + bhr[...]
    ghz = jnp.dot(hv, whz[...], preferred_element_type=jnp.float32) + bhz[...]
    ghn = jnp.dot(hv, whn[...], preferred_element_type=jnp.float32) + bhn[...]
    r = jax.nn.sigmoid(gir + ghr)
    z = jax.nn.sigmoid(giz + ghz)
    n = jnp.tanh(gin + r * ghn)
    hn = (1.0 - z) * n + z * hv
    blk = hn.shape[0]
    out[...] = jnp.concatenate(
        [hn, jnp.ones((blk, 1), jnp.float32),
         jnp.zeros((blk, 63), jnp.float32)], axis=1)


def _make_stage(np_rows: int, blk: int = 1024):
    grid = (np_rows // blk,)
    row = lambda i: (i, 0)
    full = lambda i: (0, 0)

    def spec(shape, imap):
        return pl.BlockSpec(shape, imap)

    in_specs = [
        spec((blk, 128), row),
        spec((blk, 128), row),
        spec((blk, 128), row),
        spec((64, 128), full),
        spec((64, 128), full),
        spec((1, 128), full),
        spec((128, 64), full), spec((128, 64), full), spec((128, 64), full),
        spec((64, 64), full), spec((64, 64), full), spec((64, 64), full),
        spec((1, 64), full), spec((1, 64), full), spec((1, 64), full),
        spec((1, 64), full), spec((1, 64), full), spec((1, 64), full),
    ]
    return pl.pallas_call(
        _stage_body,
        grid=grid,
        in_specs=in_specs,
        out_specs=spec((blk, 128), row),
        out_shape=jax.ShapeDtypeStruct((np_rows, 128), jnp.float32),
    )


def _dir_weights(Wm, bm, Wih, Whh, bih, bhh, l):
    wm = Wm[l]
    w1t = wm[:, :64].T
    w2t = wm[:, 64:].T
    wih_t = Wih[l].T
    whh_t = Whh[l].T
    return (w1t, w2t, bm[l].reshape(1, 128),
            wih_t[:, :64], wih_t[:, 64:128], wih_t[:, 128:],
            whh_t[:, :64], whh_t[:, 64:128], whh_t[:, 128:],
            bih[l][:64].reshape(1, 64), bih[l][64:128].reshape(1, 64),
            bih[l][128:].reshape(1, 64),
            bhh[l][:64].reshape(1, 64), bhh[l][64:128].reshape(1, 64),
            bhh[l][128:].reshape(1, 64))


def kernel(h, edge_index, Wmsg_f, bmsg_f, Wih_f, Whh_f, bih_f, bhh_f,
           Wmsg_b, bmsg_b, Wih_b, Whh_b, bih_b, bhh_b):
    n_nodes = h.shape[0]
    e = edge_index.shape[1]
    np_rows = -(-n_nodes // 2048) * 2048

    nch = -(-e // (NS * CHUNK))
    nch = -(-nch // 4) * 4
    nst = nch + 4

    src, dst = edge_index[0], edge_index[1]
    pad = jnp.full((NS * nch * CHUNK - e,), n_nodes, jnp.int32)

    def stage_idx(x):
        x3 = jnp.concatenate([x, pad]).reshape(NS, nch, CHUNK)
        x3 = jnp.pad(x3, ((0, 0), (0, 4), (0, 0)), constant_values=n_nodes)
        return x3.reshape(NS * nst, CHUNK)

    src2d = stage_idx(src)
    dst2d = stage_idx(dst)
    comb_f = jnp.stack([src2d, dst2d], axis=1)
    comb_b = jnp.stack([dst2d, src2d], axis=1)

    def ext_table(hx):
        t = jnp.zeros((np_rows, 128), jnp.float32)
        t = t.at[:n_nodes, :64].set(hx)
        t = t.at[:n_nodes, 64].set(1.0)
        return t

    hf0 = ext_table(h[:, :64])
    hb0 = ext_table(h[:, 64:])

    sc = _make_sc_pass(np_rows, nch, n_nodes)
    stage = _make_stage(np_rows)

    t0f, t0b = sc(hf0, hb0, comb_f, comb_b)
    hf1 = stage(hf0, t0f, t0f,
                *_dir_weights(Wmsg_f, bmsg_f, Wih_f, Whh_f, bih_f, bhh_f, 0))
    hb1 = stage(hb0, t0b, t0b,
                *_dir_weights(Wmsg_b, bmsg_b, Wih_b, Whh_b, bih_b, bhh_b, 0))

    t1f, t1b = sc(hf1, hb1, comb_f, comb_b)
    hf2 = stage(hf1, t1f, t0f,
                *_dir_weights(Wmsg_f, bmsg_f, Wih_f, Whh_f, bih_f, bhh_f, 1))
    hb2 = stage(hb1, t1b, t0b,
                *_dir_weights(Wmsg_b, bmsg_b, Wih_b, Whh_b, bih_b, bhh_b, 1))

    return jnp.concatenate([hf2[:n_nodes, :64], hb2[:n_nodes, :64]], axis=1)


# speedup vs baseline: 6.1171x; 1.4619x over previous
import functools

import jax
import jax.numpy as jnp
from jax import lax
from jax.experimental import pallas as pl
from jax.experimental.pallas import tpu as pltpu
from jax.experimental.pallas import tpu_sc as plsc

NC = 2
NS = 16
CHUNK = 128


def _make_sc_pass(np_rows: int, nch: int, trash: int, wl: int = 80):
    nst = nch + 4
    stripe = np_rows // NS
    n_zero = stripe // CHUNK

    mesh = plsc.VectorSubcoreMesh(
        core_axis_name="c", subcore_axis_name="s", num_cores=NC,
        num_subcores=NS)

    def body(hf, hb, cf3d, cb3d, tf, tb,
             idx_v, rows_v, acc_s,
             semg0, semg1, semi0, semi1, semi2, semi3):
        c = lax.axis_index("c")
        s = lax.axis_index("s")
        base = s * stripe
        semg = (semg0, semg1)
        semi = (semi0, semi1, semi2, semi3)

        zv = jnp.zeros((16,), jnp.float32)

        def zrow(i, _):
            for j in range(wl // 16):
                rows_v[0, i, pl.ds(j * 16, 16)] = zv
            return 0
        lax.fori_loop(0, CHUNK, zrow, 0)
        for k in range(n_zero):
            pltpu.sync_copy(rows_v.at[0],
                            acc_s.at[pl.ds(base + k * CHUNK, CHUNK)])
        plsc.subcore_barrier()

        def run(htab, c3d, t_out):
            row0 = s * nst

            def fire_idx(ch, r):
                pltpu.async_copy(c3d.at[row0 + ch], idx_v.at[r], semi[r])

            def wait_idx(ch, r):
                pltpu.make_async_copy(c3d.at[row0 + ch], idx_v.at[r],
                                      semi[r]).wait()

            def fire_g(r, b):
                pltpu.async_copy(htab.at[idx_v.at[r, 0]], rows_v.at[b],
                                 semg[b])

            def wait_g(r, b):
                pltpu.make_async_copy(htab.at[idx_v.at[r, 0]], rows_v.at[b],
                                      semg[b]).wait()

            for u in range(4):
                fire_idx(u, u)
            for u in range(2):
                wait_idx(u, u)
                fire_g(u, u)

            def step(q, _):
                cb = 4 * q
                for u in range(4):
                    ch = cb + u
                    b = u % 2
                    wait_g(u, b)
                    pltpu.sync_copy(rows_v.at[b], acc_s.at[idx_v.at[u, 1]],
                                    add=True)
                    fire_idx(ch + 4, u)
                    wait_idx(ch + 2, (u + 2) % 4)
                    fire_g((u + 2) % 4, b)
                return 0
            lax.fori_loop(0, nch // 4, step, 0)
            wait_g(0, 0)
            wait_g(1, 1)
            wait_idx(nch + 2, 2)
            wait_idx(nch + 3, 3)

            plsc.subcore_barrier()
            pltpu.sync_copy(acc_s.at[pl.ds(base, stripe)],
                            t_out.at[pl.ds(base, stripe)])

        @pl.when(c == 0)
        def _():
            run(hf, cf3d, tf)

        @pl.when(c == 1)
        def _():
            run(hb, cb3d, tb)

    out = jax.ShapeDtypeStruct((np_rows, wl), jnp.float32)
    return pl.kernel(
        body,
        out_type=(out, out),
        mesh=mesh,
        compiler_params=pltpu.CompilerParams(
            needs_layout_passes=False, use_tc_tiling_on_sc=False),
        scratch_types=[
            pltpu.VMEM((4, 2, CHUNK), jnp.int32),
            pltpu.VMEM((2, CHUNK, wl), jnp.float32),
            pltpu.VMEM_SHARED((np_rows, wl), jnp.float32),
            pltpu.SemaphoreType.DMA,
            pltpu.SemaphoreType.DMA,
            pltpu.SemaphoreType.DMA,
            pltpu.SemaphoreType.DMA,
            pltpu.SemaphoreType.DMA,
            pltpu.SemaphoreType.DMA,
        ],
    )


def _stage_body(hh, t, d, w1t, w2t, bm, wir, wiz, win,
                whr, whz, whn, bir, biz, binn, bhr, bhz, bhn, out):
    hv = hh[:, :64]
    feats = t[:, :64]
    deg = d[:, 64:65]
    aggr = (jnp.dot(feats, w1t[...], preferred_element_type=jnp.float32)
            + deg * (jnp.dot(hv, w2t[...],
                             preferred_element_type=jnp.float32) + bm[...]))
    gir = jnp.dot(aggr, wir[...], preferred_element_type=jnp.float32) + bir[...]
    giz = jnp.dot(aggr, wiz[...], preferred_element_type=jnp.float32) + biz[...]
    gin = jnp.dot(aggr, win[...], preferred_element_type=jnp.float32) + binn[...]
    ghr = jnp.dot(hv, whr[...], preferred_element_type=jnp.float32) + bhr[...]
    ghz = jnp.dot(hv, whz[...], preferred_element_type=jnp.float32) + bhz[...]
    ghn = jnp.dot(hv, whn[...], preferred_element_type=jnp.float32) + bhn[...]
    r = jax.nn.sigmoid(gir + ghr)
    z = jax.nn.sigmoid(giz + ghz)
    n = jnp.tanh(gin + r * ghn)
    hn = (1.0 - z) * n + z * hv
    blk = hn.shape[0]
    out[...] = jnp.concatenate(
        [hn, jnp.ones((blk, 1), jnp.float32),
         jnp.zeros((blk, 15), jnp.float32)], axis=1)


def _make_stage(np_rows: int, blk: int = 1024):
    grid = (np_rows // blk,)
    row = lambda i: (i, 0)
    full = lambda i: (0, 0)

    def spec(shape, imap):
        return pl.BlockSpec(shape, imap)

    in_specs = [
        spec((blk, 80), row),
        spec((blk, 80), row),
        spec((blk, 80), row),
        spec((64, 128), full),
        spec((64, 128), full),
        spec((1, 128), full),
        spec((128, 64), full), spec((128, 64), full), spec((128, 64), full),
        spec((64, 64), full), spec((64, 64), full), spec((64, 64), full),
        spec((1, 64), full), spec((1, 64), full), spec((1, 64), full),
        spec((1, 64), full), spec((1, 64), full), spec((1, 64), full),
    ]
    return pl.pallas_call(
        _stage_body,
        grid=grid,
        in_specs=in_specs,
        out_specs=spec((blk, 80), row),
        out_shape=jax.ShapeDtypeStruct((np_rows, 80), jnp.float32),
    )


def _dir_weights(Wm, bm, Wih, Whh, bih, bhh, l):
    wm = Wm[l]
    w1t = wm[:, :64].T
    w2t = wm[:, 64:].T
    wih_t = Wih[l].T
    whh_t = Whh[l].T
    return (w1t, w2t, bm[l].reshape(1, 128),
            wih_t[:, :64], wih_t[:, 64:128], wih_t[:, 128:],
            whh_t[:, :64], whh_t[:, 64:128], whh_t[:, 128:],
            bih[l][:64].reshape(1, 64), bih[l][64:128].reshape(1, 64),
            bih[l][128:].reshape(1, 64),
            bhh[l][:64].reshape(1, 64), bhh[l][64:128].reshape(1, 64),
            bhh[l][128:].reshape(1, 64))


def kernel(h, edge_index, Wmsg_f, bmsg_f, Wih_f, Whh_f, bih_f, bhh_f,
           Wmsg_b, bmsg_b, Wih_b, Whh_b, bih_b, bhh_b):
    n_nodes = h.shape[0]
    e = edge_index.shape[1]
    np_rows = -(-n_nodes // 2048) * 2048

    nch = -(-e // (NS * CHUNK))
    nch = -(-nch // 4) * 4
    nst = nch + 4

    src, dst = edge_index[0], edge_index[1]
    pad = jnp.full((NS * nch * CHUNK - e,), n_nodes, jnp.int32)

    def stage_idx(x):
        x3 = jnp.concatenate([x, pad]).reshape(NS, nch, CHUNK)
        x3 = jnp.pad(x3, ((0, 0), (0, 4), (0, 0)), constant_values=n_nodes)
        return x3.reshape(NS * nst, CHUNK)

    src2d = stage_idx(src)
    dst2d = stage_idx(dst)
    comb_f = jnp.stack([src2d, dst2d], axis=1)
    comb_b = jnp.stack([dst2d, src2d], axis=1)

    def ext_table(hx):
        t = jnp.zeros((np_rows, 80), jnp.float32)
        t = t.at[:n_nodes, :64].set(hx)
        t = t.at[:n_nodes, 64].set(1.0)
        return t

    hf0 = ext_table(h[:, :64])
    hb0 = ext_table(h[:, 64:])

    sc = _make_sc_pass(np_rows, nch, n_nodes)
    stage = _make_stage(np_rows)

    t0f, t0b = sc(hf0, hb0, comb_f, comb_b)
    hf1 = stage(hf0, t0f, t0f,
                *_dir_weights(Wmsg_f, bmsg_f, Wih_f, Whh_f, bih_f, bhh_f, 0))
    hb1 = stage(hb0, t0b, t0b,
                *_dir_weights(Wmsg_b, bmsg_b, Wih_b, Whh_b, bih_b, bhh_b, 0))

    t1f, t1b = sc(hf1, hb1, comb_f, comb_b)
    hf2 = stage(hf1, t1f, t0f,
                *_dir_weights(Wmsg_f, bmsg_f, Wih_f, Whh_f, bih_f, bhh_f, 1))
    hb2 = stage(hb1, t1b, t0b,
                *_dir_weights(Wmsg_b, bmsg_b, Wih_b, Whh_b, bih_b, bhh_b, 1))

    return jnp.concatenate([hf2[:n_nodes, :64], hb2[:n_nodes, :64]], axis=1)
